# initial kernel scaffold (unmeasured)
import functools

import jax
import jax.numpy as jnp
from jax import lax
from jax.experimental import pallas as pl
from jax.experimental.pallas import tpu as pltpu

N_DEV = 8
SQ = 2048
SKV = 2048
D_MODEL = 1024
H_PER = 8
DH = 128
SCALE = 0.08838834764831843
CHUNK = SQ // N_DEV


def _attn_body(x_ref, wq_ref, k_ref, v_ref, ctx_ref):
    xm = x_ref[0]
    q = jnp.dot(xm, wq_ref[...], preferred_element_type=jnp.float32)
    k = k_ref[0, :, 0, :]
    v = v_ref[0, :, 0, :]
    s = lax.dot_general(
        q, k, (((1,), (1,)), ((), ())), preferred_element_type=jnp.float32
    ) * SCALE
    qi = lax.broadcasted_iota(jnp.int32, (SQ, SKV), 0)
    ki = lax.broadcasted_iota(jnp.int32, (SQ, SKV), 1)
    mask = (jnp.abs(qi - ki) <= 128) | (ki < 32) | (qi < 32)
    s = jnp.where(mask, s, -1e9)
    m = jnp.max(s, axis=1, keepdims=True)
    w = jnp.exp(s - m)
    w = w / jnp.sum(w, axis=1, keepdims=True)
    ctx_ref[...] = jnp.dot(w, v, preferred_element_type=jnp.float32)


def _attention(x, Wq_l, K_ext, V_ext):
    return pl.pallas_call(
        _attn_body,
        grid=(H_PER,),
        in_specs=[
            pl.BlockSpec((1, SQ, D_MODEL), lambda h: (0, 0, 0)),
            pl.BlockSpec((D_MODEL, DH), lambda h: (0, h)),
            pl.BlockSpec((1, SKV, 1, DH), lambda h: (0, 0, h, 0)),
            pl.BlockSpec((1, SKV, 1, DH), lambda h: (0, 0, h, 0)),
        ],
        out_specs=pl.BlockSpec((SQ, DH), lambda h: (0, h)),
        out_shape=jax.ShapeDtypeStruct((SQ, H_PER * DH), jnp.float32),
    )(x, Wq_l, K_ext, V_ext)


def _allreduce_body(
    ctx_ref, wo_ref, out_ref, rs_buf, ag_buf, rs_send, rs_recv, ag_send, ag_recv
):
    my = lax.axis_index("i")
    left = lax.rem(my + N_DEV - 1, N_DEV)
    right = lax.rem(my + 1, N_DEV)

    out_ref[0] = jnp.dot(
        ctx_ref[...], wo_ref[...], preferred_element_type=jnp.float32
    )

    barrier_sem = pltpu.get_barrier_semaphore()
    for nbr in (left, right):
        pl.semaphore_signal(
            barrier_sem, inc=1,
            device_id=(nbr,), device_id_type=pl.DeviceIdType.MESH,
        )
    pl.semaphore_wait(barrier_sem, 2)

    def chunk_at(ref, c):
        return ref.at[0, pl.ds(c * CHUNK, CHUNK), :]

    for s in range(N_DEV - 1):
        sc = lax.rem(my - s + N_DEV, N_DEV)
        rc = lax.rem(my - s - 1 + N_DEV, N_DEV)
        rdma = pltpu.make_async_remote_copy(
            src_ref=chunk_at(out_ref, sc),
            dst_ref=rs_buf.at[s],
            send_sem=rs_send.at[s],
            recv_sem=rs_recv.at[s],
            device_id=(right,),
            device_id_type=pl.DeviceIdType.MESH,
        )
        rdma.start()
        rdma.wait()
        out_ref[0, pl.ds(rc * CHUNK, CHUNK), :] += rs_buf[s]

    for t in range(N_DEV - 1):
        sc = lax.rem(my + 1 - t + N_DEV, N_DEV)
        rc = lax.rem(my - t + N_DEV, N_DEV)
        rdma = pltpu.make_async_remote_copy(
            src_ref=chunk_at(out_ref, sc),
            dst_ref=ag_buf.at[t],
            send_sem=ag_send.at[t],
            recv_sem=ag_recv.at[t],
            device_id=(right,),
            device_id_type=pl.DeviceIdType.MESH,
        )
        rdma.start()
        rdma.wait()
        out_ref[0, pl.ds(rc * CHUNK, CHUNK), :] = ag_buf[t]


def _project_allreduce(ctx, Wo_l):
    return pl.pallas_call(
        _allreduce_body,
        out_shape=jax.ShapeDtypeStruct((1, SQ, D_MODEL), jnp.float32),
        in_specs=[
            pl.BlockSpec(memory_space=pltpu.VMEM),
            pl.BlockSpec(memory_space=pltpu.VMEM),
        ],
        out_specs=pl.BlockSpec(memory_space=pltpu.VMEM),
        scratch_shapes=[
            pltpu.VMEM((N_DEV - 1, CHUNK, D_MODEL), jnp.float32),
            pltpu.VMEM((N_DEV - 1, CHUNK, D_MODEL), jnp.float32),
            pltpu.SemaphoreType.DMA((N_DEV - 1,)),
            pltpu.SemaphoreType.DMA((N_DEV - 1,)),
            pltpu.SemaphoreType.DMA((N_DEV - 1,)),
            pltpu.SemaphoreType.DMA((N_DEV - 1,)),
        ],
        compiler_params=pltpu.CompilerParams(collective_id=0),
    )(ctx, Wo_l)


def kernel(x, Wq, K_ext, V_ext, Wo):
    idx = lax.axis_index("i")
    Wq_l = lax.dynamic_slice(Wq, (0, idx * H_PER * DH), (D_MODEL, H_PER * DH))
    Wo_l = lax.dynamic_slice(Wo, (idx * H_PER * DH, 0), (H_PER * DH, D_MODEL))
    ctx = _attention(x, Wq_l, K_ext, V_ext)
    return _project_allreduce(ctx, Wo_l)


# baseline (device time: 303573 ns/iter reference)
import functools

import jax
import jax.numpy as jnp
from jax import lax
from jax.experimental import pallas as pl
from jax.experimental.pallas import tpu as pltpu

N_DEV = 8
SQ = 2048
SKV = 2048
D_MODEL = 1024
H_PER = 8
DH = 128
SCALE = 0.08838834764831843
CHUNK = SQ // N_DEV


def _attn_body(x_ref, wq_ref, k_ref, v_ref, ctx_ref):
    xm = x_ref[0]
    q = jnp.dot(xm, wq_ref[...], preferred_element_type=jnp.float32)
    k = k_ref[0]
    v = v_ref[0]
    s = lax.dot_general(
        q, k, (((1,), (1,)), ((), ())), preferred_element_type=jnp.float32
    ) * SCALE
    qi = lax.broadcasted_iota(jnp.int32, (SQ, SKV), 0)
    ki = lax.broadcasted_iota(jnp.int32, (SQ, SKV), 1)
    mask = (jnp.abs(qi - ki) <= 128) | (ki < 32) | (qi < 32)
    s = jnp.where(mask, s, -1e9)
    m = jnp.max(s, axis=1, keepdims=True)
    w = jnp.exp(s - m)
    w = w / jnp.sum(w, axis=1, keepdims=True)
    ctx_ref[...] = jnp.dot(w, v, preferred_element_type=jnp.float32)


def _attention(x, Wq_l, K_ext, V_ext):
    return pl.pallas_call(
        _attn_body,
        grid=(H_PER,),
        in_specs=[
            pl.BlockSpec((1, SQ, D_MODEL), lambda h: (0, 0, 0)),
            pl.BlockSpec((D_MODEL, DH), lambda h: (0, h)),
            pl.BlockSpec((1, SKV, DH), lambda h: (h, 0, 0)),
            pl.BlockSpec((1, SKV, DH), lambda h: (h, 0, 0)),
        ],
        out_specs=pl.BlockSpec((SQ, DH), lambda h: (0, h)),
        out_shape=jax.ShapeDtypeStruct((SQ, H_PER * DH), jnp.float32),
    )(x, Wq_l, K_ext, V_ext)


def _allreduce_body(
    ctx_ref, wo_ref, out_ref, rs_buf, ag_buf, rs_send, rs_recv, ag_send, ag_recv
):
    my = lax.axis_index("i")
    left = lax.rem(my + N_DEV - 1, N_DEV)
    right = lax.rem(my + 1, N_DEV)

    out_ref[0] = jnp.dot(
        ctx_ref[...], wo_ref[...], preferred_element_type=jnp.float32
    )

    barrier_sem = pltpu.get_barrier_semaphore()
    for nbr in (left, right):
        pl.semaphore_signal(
            barrier_sem, inc=1,
            device_id=(nbr,), device_id_type=pl.DeviceIdType.MESH,
        )
    pl.semaphore_wait(barrier_sem, 2)

    def chunk_at(ref, c):
        return ref.at[0, pl.ds(c * CHUNK, CHUNK), :]

    for s in range(N_DEV - 1):
        sc = lax.rem(my - s + N_DEV, N_DEV)
        rc = lax.rem(my - s - 1 + N_DEV, N_DEV)
        rdma = pltpu.make_async_remote_copy(
            src_ref=chunk_at(out_ref, sc),
            dst_ref=rs_buf.at[s],
            send_sem=rs_send.at[s],
            recv_sem=rs_recv.at[s],
            device_id=(right,),
            device_id_type=pl.DeviceIdType.MESH,
        )
        rdma.start()
        rdma.wait()
        out_ref[0, pl.ds(rc * CHUNK, CHUNK), :] += rs_buf[s]

    for t in range(N_DEV - 1):
        sc = lax.rem(my + 1 - t + N_DEV, N_DEV)
        rc = lax.rem(my - t + N_DEV, N_DEV)
        rdma = pltpu.make_async_remote_copy(
            src_ref=chunk_at(out_ref, sc),
            dst_ref=ag_buf.at[t],
            send_sem=ag_send.at[t],
            recv_sem=ag_recv.at[t],
            device_id=(right,),
            device_id_type=pl.DeviceIdType.MESH,
        )
        rdma.start()
        rdma.wait()
        out_ref[0, pl.ds(rc * CHUNK, CHUNK), :] = ag_buf[t]


def _project_allreduce(ctx, Wo_l):
    return pl.pallas_call(
        _allreduce_body,
        out_shape=jax.ShapeDtypeStruct((1, SQ, D_MODEL), jnp.float32),
        in_specs=[
            pl.BlockSpec(memory_space=pltpu.VMEM),
            pl.BlockSpec(memory_space=pltpu.VMEM),
        ],
        out_specs=pl.BlockSpec(memory_space=pltpu.VMEM),
        scratch_shapes=[
            pltpu.VMEM((N_DEV - 1, CHUNK, D_MODEL), jnp.float32),
            pltpu.VMEM((N_DEV - 1, CHUNK, D_MODEL), jnp.float32),
            pltpu.SemaphoreType.DMA((N_DEV - 1,)),
            pltpu.SemaphoreType.DMA((N_DEV - 1,)),
            pltpu.SemaphoreType.DMA((N_DEV - 1,)),
            pltpu.SemaphoreType.DMA((N_DEV - 1,)),
        ],
        compiler_params=pltpu.CompilerParams(collective_id=0),
    )(ctx, Wo_l)


def kernel(x, Wq, K_ext, V_ext, Wo):
    idx = lax.axis_index("i")
    Wq_l = lax.dynamic_slice(Wq, (0, idx * H_PER * DH), (D_MODEL, H_PER * DH))
    Wo_l = lax.dynamic_slice(Wo, (idx * H_PER * DH, 0), (H_PER * DH, D_MODEL))
    K = jnp.transpose(K_ext[0], (1, 0, 2))
    V = jnp.transpose(V_ext[0], (1, 0, 2))
    ctx = _attention(x, Wq_l, K, V)
    return _project_allreduce(ctx, Wo_l)


# device time: 191424 ns/iter; 1.5859x vs baseline; 1.5859x over previous
import functools

import jax
import jax.numpy as jnp
from jax import lax
from jax.experimental import pallas as pl
from jax.experimental.pallas import tpu as pltpu

N_DEV = 8
SQ = 2048
SKV = 2048
D_MODEL = 1024
H_PER = 8
DH = 128
SCALE = 0.08838834764831843
CHUNK = SQ // N_DEV


def _attn_body(x_ref, wq_ref, k_ref, v_ref, ctx_ref):
    xm = x_ref[0]
    q = jnp.dot(xm, wq_ref[...], preferred_element_type=jnp.float32)
    k = k_ref[0]
    v = v_ref[0]
    s = lax.dot_general(
        q, k, (((1,), (1,)), ((), ())), preferred_element_type=jnp.float32
    ) * SCALE
    qi = lax.broadcasted_iota(jnp.int32, (SQ, SKV), 0)
    ki = lax.broadcasted_iota(jnp.int32, (SQ, SKV), 1)
    mask = (jnp.abs(qi - ki) <= 128) | (ki < 32) | (qi < 32)
    s = jnp.where(mask, s, -1e9)
    m = jnp.max(s, axis=1, keepdims=True)
    w = jnp.exp(s - m)
    w = w / jnp.sum(w, axis=1, keepdims=True)
    ctx_ref[...] = jnp.dot(w, v, preferred_element_type=jnp.float32)


def _attention(x, Wq_l, K_ext, V_ext):
    return pl.pallas_call(
        _attn_body,
        grid=(H_PER,),
        in_specs=[
            pl.BlockSpec((1, SQ, D_MODEL), lambda h: (0, 0, 0)),
            pl.BlockSpec((D_MODEL, DH), lambda h: (0, h)),
            pl.BlockSpec((1, SKV, DH), lambda h: (h, 0, 0)),
            pl.BlockSpec((1, SKV, DH), lambda h: (h, 0, 0)),
        ],
        out_specs=pl.BlockSpec((SQ, DH), lambda h: (0, h)),
        out_shape=jax.ShapeDtypeStruct((SQ, H_PER * DH), jnp.float32),
    )(x, Wq_l, K_ext, V_ext)


HCHUNK = SQ // 2 // N_DEV


def _allreduce_body(ctx_ref, wo_ref, out_ref, send_buf, recv_buf, sems):
    my = lax.axis_index("i")
    left = lax.rem(my + N_DEV - 1, N_DEV)
    right = lax.rem(my + 1, N_DEV)

    out_ref[0] = jnp.dot(
        ctx_ref[...], wo_ref[...], preferred_element_type=jnp.float32
    )

    barrier_sem = pltpu.get_barrier_semaphore()
    for nbr in (left, right):
        pl.semaphore_signal(
            barrier_sem, inc=1,
            device_id=(nbr,), device_id_type=pl.DeviceIdType.MESH,
        )
    pl.semaphore_wait(barrier_sem, 2)

    def rows(ring, c):
        return pl.ds(ring * (SQ // 2) + c * HCHUNK, HCHUNK)

    def hop(ring, phase, s, send_c, recv_c, accumulate):
        k = ring * 2 + phase
        send_buf[k, s] = out_ref[0, rows(ring, send_c), :].astype(jnp.bfloat16)
        rdma = pltpu.make_async_remote_copy(
            src_ref=send_buf.at[k, s],
            dst_ref=recv_buf.at[k, s],
            send_sem=sems.at[0, k, s],
            recv_sem=sems.at[1, k, s],
            device_id=(right if ring == 0 else left,),
            device_id_type=pl.DeviceIdType.MESH,
        )
        rdma.start()
        return rdma, (ring, recv_c, accumulate, k, s)

    def finish(pending):
        rdma, (ring, recv_c, accumulate, k, s) = pending
        rdma.wait()
        got = recv_buf[k, s].astype(jnp.float32)
        if accumulate:
            out_ref[0, rows(ring, recv_c), :] += got
        else:
            out_ref[0, rows(ring, recv_c), :] = got

    for s in range(N_DEV - 1):
        pa = hop(0, 0, s, lax.rem(my - s + N_DEV, N_DEV),
                 lax.rem(my - s - 1 + N_DEV, N_DEV), True)
        pb = hop(1, 0, s, lax.rem(my + s, N_DEV),
                 lax.rem(my + s + 1, N_DEV), True)
        finish(pa)
        finish(pb)

    for t in range(N_DEV - 1):
        pa = hop(0, 1, t, lax.rem(my + 1 - t + N_DEV, N_DEV),
                 lax.rem(my - t + N_DEV, N_DEV), False)
        pb = hop(1, 1, t, lax.rem(my - 1 + t + N_DEV, N_DEV),
                 lax.rem(my + t, N_DEV), False)
        finish(pa)
        finish(pb)


def _project_allreduce(ctx, Wo_l):
    return pl.pallas_call(
        _allreduce_body,
        out_shape=jax.ShapeDtypeStruct((1, SQ, D_MODEL), jnp.float32),
        in_specs=[
            pl.BlockSpec(memory_space=pltpu.VMEM),
            pl.BlockSpec(memory_space=pltpu.VMEM),
        ],
        out_specs=pl.BlockSpec(memory_space=pltpu.VMEM),
        scratch_shapes=[
            pltpu.VMEM((4, N_DEV - 1, HCHUNK, D_MODEL), jnp.bfloat16),
            pltpu.VMEM((4, N_DEV - 1, HCHUNK, D_MODEL), jnp.bfloat16),
            pltpu.SemaphoreType.DMA((2, 4, N_DEV - 1)),
        ],
        compiler_params=pltpu.CompilerParams(collective_id=0),
    )(ctx, Wo_l)


def kernel(x, Wq, K_ext, V_ext, Wo):
    idx = lax.axis_index("i")
    Wq_l = lax.dynamic_slice(Wq, (0, idx * H_PER * DH), (D_MODEL, H_PER * DH))
    Wo_l = lax.dynamic_slice(Wo, (idx * H_PER * DH, 0), (H_PER * DH, D_MODEL))
    K = jnp.transpose(K_ext[0], (1, 0, 2))
    V = jnp.transpose(V_ext[0], (1, 0, 2))
    ctx = _attention(x, Wq_l, K, V)
    return _project_allreduce(ctx, Wo_l)


# device time: 161951 ns/iter; 1.8745x vs baseline; 1.1820x over previous
import functools

import jax
import jax.numpy as jnp
from jax import lax
from jax.experimental import pallas as pl
from jax.experimental.pallas import tpu as pltpu

N_DEV = 8
SQ = 2048
SKV = 2048
D_MODEL = 1024
H_PER = 8
DH = 128
SCALE = 0.08838834764831843
CHUNK = SQ // N_DEV


QT = 256
N_QT = SQ // QT
KT = 256
N_KT = SKV // KT


def _key_tiles(qt: int) -> list[int]:
    if qt == 0:
        return list(range(N_KT))
    tiles = {0, qt - 1, qt, qt + 1}
    return sorted(t for t in tiles if 0 <= t < N_KT)


def _attn_body(x_ref, wq_ref, k_ref, v_ref, ctx_ref):
    xm = x_ref[0]
    q = jnp.dot(xm, wq_ref[...], preferred_element_type=jnp.float32)
    k = k_ref[...]
    v = v_ref[...]
    for qt in range(N_QT):
        tiles = _key_tiles(qt)
        q_t = q[qt * QT:(qt + 1) * QT, :]
        k_sel = jnp.concatenate([k[t * KT:(t + 1) * KT, :] for t in tiles], 0)
        v_sel = jnp.concatenate([v[t * KT:(t + 1) * KT, :] for t in tiles], 0)
        s = lax.dot_general(
            q_t, k_sel, (((1,), (1,)), ((), ())),
            preferred_element_type=jnp.float32,
        ) * SCALE
        qi = qt * QT + lax.broadcasted_iota(jnp.int32, s.shape, 0)
        ki = jnp.concatenate(
            [t * KT + lax.broadcasted_iota(jnp.int32, (QT, KT), 1) for t in tiles],
            axis=1,
        )
        mask = (jnp.abs(qi - ki) <= 128) | (ki < 32) | (qi < 32)
        s = jnp.where(mask, s, -1e9)
        m = jnp.max(s, axis=1, keepdims=True)
        w = jnp.exp(s - m)
        w = w / jnp.sum(w, axis=1, keepdims=True)
        ctx_ref[qt * QT:(qt + 1) * QT, :] = jnp.dot(
            w, v_sel, preferred_element_type=jnp.float32
        )


def _attention(x, Wq_l, K_flat, V_flat):
    return pl.pallas_call(
        _attn_body,
        grid=(H_PER,),
        in_specs=[
            pl.BlockSpec((1, SQ, D_MODEL), lambda h: (0, 0, 0)),
            pl.BlockSpec((D_MODEL, DH), lambda h: (0, h)),
            pl.BlockSpec((SKV, DH), lambda h: (0, h)),
            pl.BlockSpec((SKV, DH), lambda h: (0, h)),
        ],
        out_specs=pl.BlockSpec((SQ, DH), lambda h: (0, h)),
        out_shape=jax.ShapeDtypeStruct((SQ, H_PER * DH), jnp.float32),
    )(x, Wq_l, K_flat, V_flat)


HCHUNK = SQ // 2 // N_DEV


def _allreduce_body(ctx_ref, wo_ref, out_ref, send_buf, recv_buf, sems):
    my = lax.axis_index("i")
    left = lax.rem(my + N_DEV - 1, N_DEV)
    right = lax.rem(my + 1, N_DEV)

    out_ref[0] = jnp.dot(
        ctx_ref[...], wo_ref[...], preferred_element_type=jnp.float32
    )

    barrier_sem = pltpu.get_barrier_semaphore()
    for nbr in (left, right):
        pl.semaphore_signal(
            barrier_sem, inc=1,
            device_id=(nbr,), device_id_type=pl.DeviceIdType.MESH,
        )
    pl.semaphore_wait(barrier_sem, 2)

    def rows(ring, c):
        return pl.ds(ring * (SQ // 2) + c * HCHUNK, HCHUNK)

    def hop(ring, phase, s, send_c, recv_c, accumulate):
        k = ring * 2 + phase
        send_buf[k, s] = out_ref[0, rows(ring, send_c), :].astype(jnp.bfloat16)
        rdma = pltpu.make_async_remote_copy(
            src_ref=send_buf.at[k, s],
            dst_ref=recv_buf.at[k, s],
            send_sem=sems.at[0, k, s],
            recv_sem=sems.at[1, k, s],
            device_id=(right if ring == 0 else left,),
            device_id_type=pl.DeviceIdType.MESH,
        )
        rdma.start()
        return rdma, (ring, recv_c, accumulate, k, s)

    def finish(pending):
        rdma, (ring, recv_c, accumulate, k, s) = pending
        rdma.wait()
        got = recv_buf[k, s].astype(jnp.float32)
        if accumulate:
            out_ref[0, rows(ring, recv_c), :] += got
        else:
            out_ref[0, rows(ring, recv_c), :] = got

    for s in range(N_DEV - 1):
        pa = hop(0, 0, s, lax.rem(my - s + N_DEV, N_DEV),
                 lax.rem(my - s - 1 + N_DEV, N_DEV), True)
        pb = hop(1, 0, s, lax.rem(my + s, N_DEV),
                 lax.rem(my + s + 1, N_DEV), True)
        finish(pa)
        finish(pb)

    for t in range(N_DEV - 1):
        pa = hop(0, 1, t, lax.rem(my + 1 - t + N_DEV, N_DEV),
                 lax.rem(my - t + N_DEV, N_DEV), False)
        pb = hop(1, 1, t, lax.rem(my - 1 + t + N_DEV, N_DEV),
                 lax.rem(my + t, N_DEV), False)
        finish(pa)
        finish(pb)


def _project_allreduce(ctx, Wo_l):
    return pl.pallas_call(
        _allreduce_body,
        out_shape=jax.ShapeDtypeStruct((1, SQ, D_MODEL), jnp.float32),
        in_specs=[
            pl.BlockSpec(memory_space=pltpu.VMEM),
            pl.BlockSpec(memory_space=pltpu.VMEM),
        ],
        out_specs=pl.BlockSpec(memory_space=pltpu.VMEM),
        scratch_shapes=[
            pltpu.VMEM((4, N_DEV - 1, HCHUNK, D_MODEL), jnp.bfloat16),
            pltpu.VMEM((4, N_DEV - 1, HCHUNK, D_MODEL), jnp.bfloat16),
            pltpu.SemaphoreType.DMA((2, 4, N_DEV - 1)),
        ],
        compiler_params=pltpu.CompilerParams(collective_id=0),
    )(ctx, Wo_l)


def kernel(x, Wq, K_ext, V_ext, Wo):
    idx = lax.axis_index("i")
    Wq_l = lax.dynamic_slice(Wq, (0, idx * H_PER * DH), (D_MODEL, H_PER * DH))
    Wo_l = lax.dynamic_slice(Wo, (idx * H_PER * DH, 0), (H_PER * DH, D_MODEL))
    K = K_ext.reshape(SKV, H_PER * DH)
    V = V_ext.reshape(SKV, H_PER * DH)
    ctx = _attention(x, Wq_l, K, V)
    return _project_allreduce(ctx, Wo_l)
